# baseline (speedup 1.0000x reference)
import functools

import jax
import jax.numpy as jnp
from jax import lax
from jax.experimental import pallas as pl
from jax.experimental.pallas import tpu as pltpu
from jax.experimental.pallas import tpu_sc as plsc

EMB = 32
CW = 512
NC, NS = 2, 16
NW = NC * NS
L = 16
ET, EI = EMB // 8, 8
BI = 128


def _emb_kernel(n_seq, x_hbm, tab_hbm, out_hbm,
                idx0, idx1, rows0, rows1, trows0, trows1,
                gs0, gs1, os0, os1):
    wid = lax.axis_index("s") * NC + lax.axis_index("c")
    boff = wid * CW
    bt0 = wid * (CW // BI)
    bufs = ((idx0, rows0, trows0, gs0, os0),
            (idx1, rows1, trows1, gs1, os1))

    lane = jnp.arange(L, dtype=jnp.int32)
    ediag = [(jnp.arange(L, dtype=jnp.int32) + e) % EMB for e in range(EMB)]
    cdiag = [(e_ // EI) * (ET * EI * BI) + (e_ % EI) * BI + lane
             for e_ in ediag]

    def fire(buf, s):
        idx_v, rows_v, _, gsem, _ = bufs[buf]
        pltpu.sync_copy(x_hbm.at[s, pl.ds(boff, CW)], idx_v)
        return pltpu.async_copy(tab_hbm.at[idx_v], rows_v, gsem)

    def transpose(buf):
        _, rows_v, trows_v, _, _ = bufs[buf]

        def tbody(g, _):
            for u in range(4):
                b0 = g * 4 + u
                bvec = lane + b0 * L
                off = (b0 // (BI // L)) * (EI * BI) + (b0 % (BI // L)) * L
                for e in range(EMB):
                    v = plsc.load_gather(rows_v, [bvec, ediag[e]])
                    plsc.store_scatter(trows_v, [cdiag[e] + off], v)
            return _

        lax.fori_loop(0, CW // L // 4, tbody, None)

    wchunk = (CW // BI) * EI * BI

    def write(buf, s):
        _, _, trows_v, _, osem = bufs[buf]
        for et in range(ET):
            pltpu.async_copy(
                trows_v.at[pl.ds(et * wchunk, wchunk)],
                out_hbm.at[s, et, pl.ds(bt0 * (EI * BI), wchunk)], osem)

    def wait_write(buf):
        _, _, trows_v, _, osem = bufs[buf]
        for et in range(ET):
            pltpu.make_async_copy(
                trows_v.at[pl.ds(et * wchunk, wchunk)],
                out_hbm.at[0, et, pl.ds(bt0 * (EI * BI), wchunk)],
                osem).wait()

    def wait_gather(buf):
        idx_v, rows_v, _, gsem, _ = bufs[buf]
        pltpu.make_async_copy(tab_hbm.at[idx_v], rows_v, gsem).wait()

    n_pairs = n_seq // 2

    def step(i, buf, s):
        pl.when(i > 0)(lambda: wait_write(buf))
        wait_gather(buf)
        transpose(buf)
        write(buf, s)
        def refire():
            fire(buf, s + 2)

        pl.when(i < n_pairs - 1)(refire)

    def body(i, _):
        s0 = 2 * i
        step(i, 0, s0)
        step(i, 1, s0 + 1)
        return _

    fire(0, 0)
    fire(1, 1)
    lax.fori_loop(0, n_pairs, body, None)
    wait_write(0)
    wait_write(1)


@jax.jit
def _emb(xt, table4):
    n_seq, n_batch = xt.shape
    mesh = plsc.VectorSubcoreMesh(core_axis_name="c", subcore_axis_name="s",
                                  num_cores=NC, num_subcores=NS)
    k = pl.kernel(
        functools.partial(_emb_kernel, n_seq),
        out_type=jax.ShapeDtypeStruct((n_seq, ET, (n_batch // BI) * EI * BI),
                                      jnp.float32),
        mesh=mesh,
        scratch_types=[
            pltpu.VMEM((CW,), jnp.int32),
            pltpu.VMEM((CW,), jnp.int32),
            pltpu.VMEM((CW, EMB), jnp.float32),
            pltpu.VMEM((CW, EMB), jnp.float32),
            pltpu.VMEM((ET * (CW // BI) * EI * BI,), jnp.float32),
            pltpu.VMEM((ET * (CW // BI) * EI * BI,), jnp.float32),
            pltpu.SemaphoreType.DMA,
            pltpu.SemaphoreType.DMA,
            pltpu.SemaphoreType.DMA,
            pltpu.SemaphoreType.DMA,
        ],
        compiler_params=pltpu.CompilerParams(use_tc_tiling_on_sc=False,
                                             needs_layout_passes=False),
    )
    return k(xt, table4)


def kernel(x, table):
    n_batch, n_seq = x.shape
    table4 = jnp.pad(table, ((0, 0), (0, 128 - EMB))).reshape(-1, EMB)
    out5 = _emb(x.T * 4, table4).reshape(n_seq, ET, n_batch // BI, EI, BI)
    return out5.transpose(2, 4, 0, 1, 3).reshape(n_batch, n_seq, EMB)


# speedup vs baseline: 1.0176x; 1.0176x over previous
import functools

import jax
import jax.numpy as jnp
from jax import lax
from jax.experimental import pallas as pl
from jax.experimental.pallas import tpu as pltpu
from jax.experimental.pallas import tpu_sc as plsc

EMB = 32
CW = 512
NC, NS = 2, 16
NW = NC * NS
L = 16
ET, EI = EMB // 8, 8
BI = 128


def _emb_kernel(n_seq, x_hbm, tab_hbm, out_hbm,
                idx0, idx1, rows0, rows1, trows0, trows1,
                gs0, gs1, os0, os1):
    wid = lax.axis_index("s") * NC + lax.axis_index("c")
    boff = wid * CW
    bt0 = wid * (CW // BI)
    bufs = ((idx0, rows0, trows0, gs0, os0),
            (idx1, rows1, trows1, gs1, os1))

    lane = jnp.arange(L, dtype=jnp.int32)
    ediag = [(jnp.arange(L, dtype=jnp.int32) + e) % EMB for e in range(EMB)]
    cdiag = [(e_ // EI) * (ET * EI * BI) + (e_ % EI) * BI + lane
             for e_ in ediag]

    def fire(buf, s):
        idx_v, rows_v, _, gsem, _ = bufs[buf]
        pltpu.sync_copy(x_hbm.at[s, pl.ds(boff, CW)], idx_v)
        return pltpu.async_copy(tab_hbm.at[idx_v], rows_v, gsem)

    def transpose(buf):
        _, rows_v, trows_v, _, _ = bufs[buf]

        def tbody(g, _):
            for u in range(2):
                b0 = g * 2 + u
                bvec = lane + b0 * L
                off = (b0 // (BI // L)) * (EI * BI) + (b0 % (BI // L)) * L
                for e in range(EMB):
                    v = plsc.load_gather(rows_v, [bvec, ediag[e]])
                    plsc.store_scatter(trows_v, [cdiag[e] + off], v)
            return _

        lax.fori_loop(0, CW // L // 2, tbody, None)

    wchunk = (CW // BI) * EI * BI

    def write(buf, s):
        _, _, trows_v, _, osem = bufs[buf]
        for et in range(ET):
            pltpu.async_copy(
                trows_v.at[pl.ds(et * wchunk, wchunk)],
                out_hbm.at[s, et, pl.ds(bt0 * (EI * BI), wchunk)], osem)

    def wait_write(buf):
        _, _, trows_v, _, osem = bufs[buf]
        for et in range(ET):
            pltpu.make_async_copy(
                trows_v.at[pl.ds(et * wchunk, wchunk)],
                out_hbm.at[0, et, pl.ds(bt0 * (EI * BI), wchunk)],
                osem).wait()

    def wait_gather(buf):
        idx_v, rows_v, _, gsem, _ = bufs[buf]
        pltpu.make_async_copy(tab_hbm.at[idx_v], rows_v, gsem).wait()

    n_pairs = n_seq // 2

    def step(i, buf, s):
        pl.when(i > 0)(lambda: wait_write(buf))
        wait_gather(buf)
        transpose(buf)
        write(buf, s)
        def refire():
            fire(buf, s + 2)

        pl.when(i < n_pairs - 1)(refire)

    def body(i, _):
        s0 = 2 * i
        step(i, 0, s0)
        step(i, 1, s0 + 1)
        return _

    fire(0, 0)
    fire(1, 1)
    lax.fori_loop(0, n_pairs, body, None)
    wait_write(0)
    wait_write(1)


@jax.jit
def _emb(xt, table4):
    n_seq, n_batch = xt.shape
    mesh = plsc.VectorSubcoreMesh(core_axis_name="c", subcore_axis_name="s",
                                  num_cores=NC, num_subcores=NS)
    k = pl.kernel(
        functools.partial(_emb_kernel, n_seq),
        out_type=jax.ShapeDtypeStruct((n_seq, ET, (n_batch // BI) * EI * BI),
                                      jnp.float32),
        mesh=mesh,
        scratch_types=[
            pltpu.VMEM((CW,), jnp.int32),
            pltpu.VMEM((CW,), jnp.int32),
            pltpu.VMEM((CW, EMB), jnp.float32),
            pltpu.VMEM((CW, EMB), jnp.float32),
            pltpu.VMEM((ET * (CW // BI) * EI * BI,), jnp.float32),
            pltpu.VMEM((ET * (CW // BI) * EI * BI,), jnp.float32),
            pltpu.SemaphoreType.DMA,
            pltpu.SemaphoreType.DMA,
            pltpu.SemaphoreType.DMA,
            pltpu.SemaphoreType.DMA,
        ],
        compiler_params=pltpu.CompilerParams(use_tc_tiling_on_sc=False,
                                             needs_layout_passes=False),
    )
    return k(xt, table4)


def kernel(x, table):
    n_batch, n_seq = x.shape
    table4 = jnp.pad(table, ((0, 0), (0, 128 - EMB))).reshape(-1, EMB)
    out5 = _emb(x.T * 4, table4).reshape(n_seq, ET, n_batch // BI, EI, BI)
    return out5.transpose(2, 4, 0, 1, 3).reshape(n_batch, n_seq, EMB)
